# baseline (speedup 1.0000x reference)
import jax
import jax.numpy as jnp
from jax import lax
from jax.experimental import pallas as pl
from jax.experimental.pallas import tpu as pltpu
from jax.experimental.pallas import tpu_sc as plsc

VOCAB = 100000
EMB = 16
BATCH = 16384
NC = 2
NS = 16
NW = NC * NS
BPW = BATCH // NW
LANES = 16
IDXCH = 128
NBLK = BPW // IDXCH
CPB = IDXCH // LANES

TRN = 3120
TRL = VOCAB - 31 * TRN

OFF_W1 = 0
OFF_B1 = 1024
OFF_W2 = 1056
OFF_B2 = 1088
WPACK = 1104

_SC_PARAMS = pltpu.CompilerParams(
    needs_layout_passes=False, use_tc_tiling_on_sc=False)
_MESH = dict(core_axis_name="c", subcore_axis_name="s",
             num_cores=NC, num_subcores=NS)


def _tr_body(tf_hbm, out_hbm, stage, obuf, sem):
    c = lax.axis_index("c")
    s = lax.axis_index("s")
    wid = s * NC + c
    base = pl.multiple_of(wid * TRN, 8)
    riota = lax.iota(jnp.int32, LANES)

    def load_slab(nrows, off):
        cps = [pltpu.async_copy(
            tf_hbm.at[pl.ds(pl.multiple_of(k * VOCAB + base + off, 8), nrows)],
            stage.at[k].at[pl.ds(off, nrows)], sem) for k in range(EMB)]
        for cp in cps:
            cp.wait()

    def transpose(ngroups):
        def body(g, carry):
            rows = g * LANES + riota
            for k in range(EMB):
                v = stage[k, pl.ds(g * LANES, LANES)]
                plsc.store_scatter(obuf, [rows, jnp.full((LANES,), k,
                                                         jnp.int32)], v)
            return carry
        lax.fori_loop(0, ngroups, body, 0)

    @pl.when(wid < 31)
    def _main():
        load_slab(TRN, 0)
        transpose(TRN // LANES)
        pltpu.sync_copy(obuf.at[pl.ds(0, TRN)], out_hbm.at[pl.ds(base, TRN)])

    @pl.when(wid == 31)
    def _tail():
        load_slab(TRN, 0)
        load_slab(TRL - TRN, TRN)
        transpose(TRL // LANES)
        pltpu.sync_copy(obuf, out_hbm.at[pl.ds(31 * TRN, TRL)])


def _stage_ids(id_hbm, wid, idx, row):
    pltpu.sync_copy(id_hbm.at[pl.ds(pl.multiple_of(wid * BPW, 8), BPW)], idx)
    for j in range(NBLK):
        for o in range(CPB):
            v = idx[pl.ds(j * IDXCH + o * LANES, LANES)]
            row[j, pl.ds(o * LANES, LANES)] = v


def _user_body(du_hbm, ut_hbm, wp_hbm, hb_hbm, idx, row, buf, hbuf, wv, sem):
    c = lax.axis_index("c")
    s = lax.axis_index("s")
    wid = s * NC + c

    pltpu.sync_copy(wp_hbm, wv)
    _stage_ids(du_hbm, wid, idx, row)
    cps = [pltpu.async_copy(ut_hbm.at[row.at[j]],
                            buf.at[pl.ds(j * IDXCH, IDXCH)], sem)
           for j in range(NBLK)]

    riota = lax.iota(jnp.int32, LANES)
    cols = [jnp.full((LANES,), k, jnp.int32) for k in range(EMB)]

    for j in range(NBLK):
        cps[j].wait()

        def pair(ci, carry):
            base = (j * CPB + 2 * ci) * LANES
            ra = base + riota
            rb = base + LANES + riota
            fa = [plsc.load_gather(buf, [ra, cols[k]]) for k in range(EMB)]
            fb = [plsc.load_gather(buf, [rb, cols[k]]) for k in range(EMB)]
            b1a = wv[pl.ds(OFF_B1, LANES)]
            b1b = wv[pl.ds(OFF_B1 + LANES, LANES)]
            for jf in range(32):
                wa = wv[pl.ds(OFF_W1 + jf * 32, LANES)]
                bj = b1a[jf] if jf < LANES else b1b[jf - LANES]
                ha = jnp.full((LANES,), 0.0, jnp.float32) + bj
                hb = jnp.full((LANES,), 0.0, jnp.float32) + bj
                for k in range(EMB):
                    w = wa[k]
                    ha = ha + fa[k] * w
                    hb = hb + fb[k] * w
                hbuf[jf, pl.ds(base, LANES)] = ha
                hbuf[jf, pl.ds(base + LANES, LANES)] = hb
            return carry

        lax.fori_loop(0, CPB // 2, pair, 0)

    pltpu.sync_copy(hbuf, hb_hbm.at[wid])


def _item_body(di_hbm, it_hbm, wp_hbm, hb_hbm, out_hbm,
               idx, row, buf, hbuf, wv, logits_v, sem, sem2):
    c = lax.axis_index("c")
    s = lax.axis_index("s")
    wid = s * NC + c

    pltpu.sync_copy(wp_hbm, wv)
    hb_cp = pltpu.async_copy(hb_hbm.at[wid], hbuf, sem2)
    _stage_ids(di_hbm, wid, idx, row)
    cps = [pltpu.async_copy(it_hbm.at[row.at[j]],
                            buf.at[pl.ds(j * IDXCH, IDXCH)], sem)
           for j in range(NBLK)]

    riota = lax.iota(jnp.int32, LANES)
    cols = [jnp.full((LANES,), k, jnp.int32) for k in range(EMB)]

    hb_cp.wait()
    for j in range(NBLK):
        cps[j].wait()

        def pair(ci, carry):
            base = (j * CPB + 2 * ci) * LANES
            ra = base + riota
            rb = base + LANES + riota
            fa = [plsc.load_gather(buf, [ra, cols[k]]) for k in range(EMB)]
            fb = [plsc.load_gather(buf, [rb, cols[k]]) for k in range(EMB)]
            w2a = wv[pl.ds(OFF_W2, LANES)]
            w2b = wv[pl.ds(OFF_W2 + LANES, LANES)]
            b2v = wv[pl.ds(OFF_B2, LANES)]
            acc_a = jnp.full((LANES,), 0.0, jnp.float32)
            acc_b = jnp.full((LANES,), 0.0, jnp.float32)
            for jf in range(32):
                wb = wv[pl.ds(OFF_W1 + jf * 32 + LANES, LANES)]
                ha = hbuf[jf, pl.ds(base, LANES)]
                hb = hbuf[jf, pl.ds(base + LANES, LANES)]
                for k in range(EMB):
                    w = wb[k]
                    ha = ha + fa[k] * w
                    hb = hb + fb[k] * w
                ha = jnp.maximum(ha, 0.0)
                hb = jnp.maximum(hb, 0.0)
                w2j = w2a[jf] if jf < LANES else w2b[jf - LANES]
                acc_a = acc_a + ha * w2j
                acc_b = acc_b + hb * w2j
            logits_v[pl.ds(base, LANES)] = acc_a + b2v[0]
            logits_v[pl.ds(base + LANES, LANES)] = acc_b + b2v[0]
            return carry

        lax.fori_loop(0, CPB // 2, pair, 0)

    pltpu.sync_copy(
        logits_v, out_hbm.at[pl.ds(pl.multiple_of(wid * BPW, 8), BPW)])


@jax.jit
def _run(du, di, utf, itf, wpack):
    tr_f = pl.kernel(
        _tr_body,
        out_type=jax.ShapeDtypeStruct((VOCAB, EMB), jnp.float32),
        mesh=plsc.VectorSubcoreMesh(**_MESH),
        compiler_params=_SC_PARAMS,
        scratch_types=[
            pltpu.VMEM((EMB, TRL), jnp.float32),
            pltpu.VMEM((TRL, EMB), jnp.float32),
            pltpu.SemaphoreType.DMA,
        ],
    )
    ut = tr_f(utf)
    it = tr_f(itf)
    user_f = pl.kernel(
        _user_body,
        out_type=jax.ShapeDtypeStruct((NW, 32, BPW), jnp.float32),
        mesh=plsc.VectorSubcoreMesh(**_MESH),
        compiler_params=_SC_PARAMS,
        scratch_types=[
            pltpu.VMEM((BPW,), jnp.int32),
            pltpu.VMEM((NBLK, IDXCH), jnp.int32),
            pltpu.VMEM((BPW, EMB), jnp.float32),
            pltpu.VMEM((32, BPW), jnp.float32),
            pltpu.VMEM((WPACK,), jnp.float32),
            pltpu.SemaphoreType.DMA,
        ],
    )
    hb = user_f(du, ut, wpack)
    item_f = pl.kernel(
        _item_body,
        out_type=jax.ShapeDtypeStruct((BATCH,), jnp.float32),
        mesh=plsc.VectorSubcoreMesh(**_MESH),
        compiler_params=_SC_PARAMS,
        scratch_types=[
            pltpu.VMEM((BPW,), jnp.int32),
            pltpu.VMEM((NBLK, IDXCH), jnp.int32),
            pltpu.VMEM((BPW, EMB), jnp.float32),
            pltpu.VMEM((32, BPW), jnp.float32),
            pltpu.VMEM((WPACK,), jnp.float32),
            pltpu.VMEM((BPW,), jnp.float32),
            pltpu.SemaphoreType.DMA,
            pltpu.SemaphoreType.DMA,
        ],
    )
    return item_f(di, it, wpack, hb)


def kernel(dataUser, dataItem, user_table, item_table, W1, b1, W2, b2):
    du = dataUser.astype(jnp.int32)
    di = dataItem.astype(jnp.int32)
    utf = user_table.T.reshape(-1)
    itf = item_table.T.reshape(-1)
    wpack = jnp.concatenate([
        W1.reshape(-1), b1.reshape(-1), W2.reshape(-1), b2.reshape(-1),
        jnp.zeros((WPACK - (OFF_B2 + 1),), jnp.float32)])
    out = _run(du, di, utf, itf, wpack)
    return out.reshape(BATCH, 1)


# speedup vs baseline: 1.1593x; 1.1593x over previous
import jax
import jax.numpy as jnp
from jax import lax
from jax.experimental import pallas as pl
from jax.experimental.pallas import tpu as pltpu
from jax.experimental.pallas import tpu_sc as plsc

VOCAB = 100000
EMB = 16
BATCH = 16384
NC = 2
NS = 16
NW = NC * NS
BPW = BATCH // NW
LANES = 16
IDXCH = 128
NBLK = BPW // IDXCH
CPB = IDXCH // LANES

OFF_W1 = 0
OFF_B1 = 1024
OFF_W2 = 1056
OFF_B2 = 1088
WPACK = 1104

_SC_PARAMS = pltpu.CompilerParams(
    needs_layout_passes=False, use_tc_tiling_on_sc=False)
_MESH = dict(core_axis_name="c", subcore_axis_name="s",
             num_cores=NC, num_subcores=NS)


def _stage_eidx(id_hbm, wid, idx, eidx):
    pltpu.sync_copy(id_hbm.at[pl.ds(pl.multiple_of(wid * BPW, 8), BPW)], idx)
    for j in range(NBLK):
        for o in range(CPB):
            v = idx[pl.ds(j * IDXCH + o * LANES, LANES)]
            for k in range(EMB):
                eidx[pl.ds((j * EMB + k) * IDXCH + o * LANES, LANES)] = (
                    v + k * VOCAB)


def _fire_gathers(tab_flat, eidx, buf, sem):
    return [pltpu.async_copy(tab_flat.at[eidx.at[pl.ds(r * IDXCH, IDXCH)]],
                             buf.at[pl.ds(r * IDXCH, IDXCH)], sem)
            for r in range(NBLK * EMB)]


def _user_body(du_hbm, ut_flat, wp_hbm, hb_hbm, idx, eidx, buf, hbuf, wv, sem):
    c = lax.axis_index("c")
    s = lax.axis_index("s")
    wid = s * NC + c

    pltpu.sync_copy(wp_hbm, wv)
    _stage_eidx(du_hbm, wid, idx, eidx)
    cps = _fire_gathers(ut_flat, eidx, buf, sem)

    for j in range(NBLK):
        for k in range(EMB):
            cps[j * EMB + k].wait()

        def pair(ci, carry):
            la = 2 * ci * LANES
            lb = la + LANES
            fa = [buf[pl.ds((j * EMB + k) * IDXCH + la, LANES)]
                  for k in range(EMB)]
            fb = [buf[pl.ds((j * EMB + k) * IDXCH + lb, LANES)]
                  for k in range(EMB)]
            b1a = wv[pl.ds(OFF_B1, LANES)]
            b1b = wv[pl.ds(OFF_B1 + LANES, LANES)]
            for jf in range(32):
                wa = wv[pl.ds(OFF_W1 + jf * 32, LANES)]
                bj = b1a[jf] if jf < LANES else b1b[jf - LANES]
                ha = jnp.full((LANES,), 0.0, jnp.float32) + bj
                hb = jnp.full((LANES,), 0.0, jnp.float32) + bj
                for k in range(EMB):
                    w = wa[k]
                    ha = ha + fa[k] * w
                    hb = hb + fb[k] * w
                hbuf[jf, pl.ds(j * IDXCH + la, LANES)] = ha
                hbuf[jf, pl.ds(j * IDXCH + lb, LANES)] = hb
            return carry

        lax.fori_loop(0, CPB // 2, pair, 0)

    pltpu.sync_copy(hbuf, hb_hbm.at[wid])


def _item_body(di_hbm, it_flat, wp_hbm, hb_hbm, out_hbm,
               idx, eidx, buf, hbuf, wv, logits_v, sem, sem2):
    c = lax.axis_index("c")
    s = lax.axis_index("s")
    wid = s * NC + c

    pltpu.sync_copy(wp_hbm, wv)
    hb_cp = pltpu.async_copy(hb_hbm.at[wid], hbuf, sem2)
    _stage_eidx(di_hbm, wid, idx, eidx)
    cps = _fire_gathers(it_flat, eidx, buf, sem)

    hb_cp.wait()
    for j in range(NBLK):
        for k in range(EMB):
            cps[j * EMB + k].wait()

        def pair(ci, carry):
            la = 2 * ci * LANES
            lb = la + LANES
            fa = [buf[pl.ds((j * EMB + k) * IDXCH + la, LANES)]
                  for k in range(EMB)]
            fb = [buf[pl.ds((j * EMB + k) * IDXCH + lb, LANES)]
                  for k in range(EMB)]
            w2a = wv[pl.ds(OFF_W2, LANES)]
            w2b = wv[pl.ds(OFF_W2 + LANES, LANES)]
            b2v = wv[pl.ds(OFF_B2, LANES)]
            acc_a = jnp.full((LANES,), 0.0, jnp.float32)
            acc_b = jnp.full((LANES,), 0.0, jnp.float32)
            for jf in range(32):
                wb = wv[pl.ds(OFF_W1 + jf * 32 + LANES, LANES)]
                ha = hbuf[jf, pl.ds(j * IDXCH + la, LANES)]
                hb = hbuf[jf, pl.ds(j * IDXCH + lb, LANES)]
                for k in range(EMB):
                    w = wb[k]
                    ha = ha + fa[k] * w
                    hb = hb + fb[k] * w
                ha = jnp.maximum(ha, 0.0)
                hb = jnp.maximum(hb, 0.0)
                w2j = w2a[jf] if jf < LANES else w2b[jf - LANES]
                acc_a = acc_a + ha * w2j
                acc_b = acc_b + hb * w2j
            logits_v[pl.ds(j * IDXCH + la, LANES)] = acc_a + b2v[0]
            logits_v[pl.ds(j * IDXCH + lb, LANES)] = acc_b + b2v[0]
            return carry

        lax.fori_loop(0, CPB // 2, pair, 0)

    pltpu.sync_copy(
        logits_v, out_hbm.at[pl.ds(pl.multiple_of(wid * BPW, 8), BPW)])


@jax.jit
def _run(du, di, utf, itf, wpack):
    user_f = pl.kernel(
        _user_body,
        out_type=jax.ShapeDtypeStruct((NW, 32, BPW), jnp.float32),
        mesh=plsc.VectorSubcoreMesh(**_MESH),
        compiler_params=_SC_PARAMS,
        scratch_types=[
            pltpu.VMEM((BPW,), jnp.int32),
            pltpu.VMEM((NBLK * EMB * IDXCH,), jnp.int32),
            pltpu.VMEM((NBLK * EMB * IDXCH,), jnp.float32),
            pltpu.VMEM((32, BPW), jnp.float32),
            pltpu.VMEM((WPACK,), jnp.float32),
            pltpu.SemaphoreType.DMA,
        ],
    )
    hb = user_f(du, utf, wpack)
    item_f = pl.kernel(
        _item_body,
        out_type=jax.ShapeDtypeStruct((BATCH,), jnp.float32),
        mesh=plsc.VectorSubcoreMesh(**_MESH),
        compiler_params=_SC_PARAMS,
        scratch_types=[
            pltpu.VMEM((BPW,), jnp.int32),
            pltpu.VMEM((NBLK * EMB * IDXCH,), jnp.int32),
            pltpu.VMEM((NBLK * EMB * IDXCH,), jnp.float32),
            pltpu.VMEM((32, BPW), jnp.float32),
            pltpu.VMEM((WPACK,), jnp.float32),
            pltpu.VMEM((BPW,), jnp.float32),
            pltpu.SemaphoreType.DMA,
            pltpu.SemaphoreType.DMA,
        ],
    )
    return item_f(di, itf, wpack, hb)


def kernel(dataUser, dataItem, user_table, item_table, W1, b1, W2, b2):
    du = dataUser.astype(jnp.int32)
    di = dataItem.astype(jnp.int32)
    utf = user_table.T.reshape(-1)
    itf = item_table.T.reshape(-1)
    wpack = jnp.concatenate([
        W1.reshape(-1), b1.reshape(-1), W2.reshape(-1), b2.reshape(-1),
        jnp.zeros((WPACK - (OFF_B2 + 1),), jnp.float32)])
    out = _run(du, di, utf, itf, wpack)
    return out.reshape(BATCH, 1)


# speedup vs baseline: 1.1640x; 1.0041x over previous
import jax
import jax.numpy as jnp
from jax import lax
from jax.experimental import pallas as pl
from jax.experimental.pallas import tpu as pltpu
from jax.experimental.pallas import tpu_sc as plsc

VOCAB = 100000
EMB = 16
BATCH = 16384
NC = 2
NS = 16
NW = NC * NS
BPW = BATCH // NW
LANES = 16
IDXCH = 128
NBLK = BPW // IDXCH
CPB = IDXCH // LANES

OFF_W1 = 0
OFF_B1 = 1024
OFF_W2 = 1056
OFF_B2 = 1088
WPACK = 1104

_SC_PARAMS = pltpu.CompilerParams(
    needs_layout_passes=False, use_tc_tiling_on_sc=False)
_MESH = dict(core_axis_name="c", subcore_axis_name="s",
             num_cores=NC, num_subcores=NS)


def _stage_eidx(id_hbm, wid, idx, eidx):
    pltpu.sync_copy(id_hbm.at[pl.ds(pl.multiple_of(wid * BPW, 8), BPW)], idx)

    def expand(k, carry):
        for j in range(NBLK):
            for o in range(CPB):
                v = idx[pl.ds(j * IDXCH + o * LANES, LANES)]
                eidx[pl.ds((j * EMB + k) * IDXCH + o * LANES, LANES)] = (
                    v + k * VOCAB)
        return carry

    lax.fori_loop(0, EMB, expand, 0)


def _fire_gathers(tab_flat, eidx, buf, sem):
    return [pltpu.async_copy(tab_flat.at[eidx.at[pl.ds(r * IDXCH, IDXCH)]],
                             buf.at[pl.ds(r * IDXCH, IDXCH)], sem)
            for r in range(NBLK * EMB)]


def _user_body(du_hbm, ut_flat, wp_hbm, hb_hbm, idx, eidx, buf, hbuf, wv, sem):
    c = lax.axis_index("c")
    s = lax.axis_index("s")
    wid = s * NC + c

    _stage_eidx(du_hbm, wid, idx, eidx)
    cps = _fire_gathers(ut_flat, eidx, buf, sem)
    pltpu.sync_copy(wp_hbm, wv)

    for j in range(NBLK):
        for k in range(EMB):
            cps[j * EMB + k].wait()

        def pair(ci, carry):
            la = 2 * ci * LANES
            lb = la + LANES
            fa = [buf[pl.ds((j * EMB + k) * IDXCH + la, LANES)]
                  for k in range(EMB)]
            fb = [buf[pl.ds((j * EMB + k) * IDXCH + lb, LANES)]
                  for k in range(EMB)]
            b1a = wv[pl.ds(OFF_B1, LANES)]
            b1b = wv[pl.ds(OFF_B1 + LANES, LANES)]
            for jf in range(32):
                wa = wv[pl.ds(OFF_W1 + jf * 32, LANES)]
                bj = b1a[jf] if jf < LANES else b1b[jf - LANES]
                ha = jnp.full((LANES,), 0.0, jnp.float32) + bj
                hb = jnp.full((LANES,), 0.0, jnp.float32) + bj
                for k in range(EMB):
                    w = wa[k]
                    ha = ha + fa[k] * w
                    hb = hb + fb[k] * w
                hbuf[jf, pl.ds(j * IDXCH + la, LANES)] = ha
                hbuf[jf, pl.ds(j * IDXCH + lb, LANES)] = hb
            return carry

        lax.fori_loop(0, CPB // 2, pair, 0)

    pltpu.sync_copy(hbuf, hb_hbm.at[wid])


def _item_body(di_hbm, it_flat, wp_hbm, hb_hbm, out_hbm,
               idx, eidx, buf, hbuf, wv, logits_v, sem, sem2):
    c = lax.axis_index("c")
    s = lax.axis_index("s")
    wid = s * NC + c

    _stage_eidx(di_hbm, wid, idx, eidx)
    cps = _fire_gathers(it_flat, eidx, buf, sem)
    pltpu.sync_copy(wp_hbm, wv)
    hb_cp = pltpu.async_copy(hb_hbm.at[wid], hbuf, sem2)

    hb_cp.wait()
    for j in range(NBLK):
        for k in range(EMB):
            cps[j * EMB + k].wait()

        def pair(ci, carry):
            la = 2 * ci * LANES
            lb = la + LANES
            fa = [buf[pl.ds((j * EMB + k) * IDXCH + la, LANES)]
                  for k in range(EMB)]
            fb = [buf[pl.ds((j * EMB + k) * IDXCH + lb, LANES)]
                  for k in range(EMB)]
            w2a = wv[pl.ds(OFF_W2, LANES)]
            w2b = wv[pl.ds(OFF_W2 + LANES, LANES)]
            b2v = wv[pl.ds(OFF_B2, LANES)]
            acc_a = jnp.full((LANES,), 0.0, jnp.float32)
            acc_b = jnp.full((LANES,), 0.0, jnp.float32)
            for jf in range(32):
                wb = wv[pl.ds(OFF_W1 + jf * 32 + LANES, LANES)]
                ha = hbuf[jf, pl.ds(j * IDXCH + la, LANES)]
                hb = hbuf[jf, pl.ds(j * IDXCH + lb, LANES)]
                for k in range(EMB):
                    w = wb[k]
                    ha = ha + fa[k] * w
                    hb = hb + fb[k] * w
                ha = jnp.maximum(ha, 0.0)
                hb = jnp.maximum(hb, 0.0)
                w2j = w2a[jf] if jf < LANES else w2b[jf - LANES]
                acc_a = acc_a + ha * w2j
                acc_b = acc_b + hb * w2j
            logits_v[pl.ds(j * IDXCH + la, LANES)] = acc_a + b2v[0]
            logits_v[pl.ds(j * IDXCH + lb, LANES)] = acc_b + b2v[0]
            return carry

        lax.fori_loop(0, CPB // 2, pair, 0)

    pltpu.sync_copy(
        logits_v, out_hbm.at[pl.ds(pl.multiple_of(wid * BPW, 8), BPW)])


@jax.jit
def _run(du, di, utf, itf, wpack):
    user_f = pl.kernel(
        _user_body,
        out_type=jax.ShapeDtypeStruct((NW, 32, BPW), jnp.float32),
        mesh=plsc.VectorSubcoreMesh(**_MESH),
        compiler_params=_SC_PARAMS,
        scratch_types=[
            pltpu.VMEM((BPW,), jnp.int32),
            pltpu.VMEM((NBLK * EMB * IDXCH,), jnp.int32),
            pltpu.VMEM((NBLK * EMB * IDXCH,), jnp.float32),
            pltpu.VMEM((32, BPW), jnp.float32),
            pltpu.VMEM((WPACK,), jnp.float32),
            pltpu.SemaphoreType.DMA,
        ],
    )
    hb = user_f(du, utf, wpack)
    item_f = pl.kernel(
        _item_body,
        out_type=jax.ShapeDtypeStruct((BATCH,), jnp.float32),
        mesh=plsc.VectorSubcoreMesh(**_MESH),
        compiler_params=_SC_PARAMS,
        scratch_types=[
            pltpu.VMEM((BPW,), jnp.int32),
            pltpu.VMEM((NBLK * EMB * IDXCH,), jnp.int32),
            pltpu.VMEM((NBLK * EMB * IDXCH,), jnp.float32),
            pltpu.VMEM((32, BPW), jnp.float32),
            pltpu.VMEM((WPACK,), jnp.float32),
            pltpu.VMEM((BPW,), jnp.float32),
            pltpu.SemaphoreType.DMA,
            pltpu.SemaphoreType.DMA,
        ],
    )
    return item_f(di, itf, wpack, hb)


def kernel(dataUser, dataItem, user_table, item_table, W1, b1, W2, b2):
    du = dataUser.astype(jnp.int32)
    di = dataItem.astype(jnp.int32)
    utf = user_table.T.reshape(-1)
    itf = item_table.T.reshape(-1)
    wpack = jnp.concatenate([
        W1.reshape(-1), b1.reshape(-1), W2.reshape(-1), b2.reshape(-1),
        jnp.zeros((WPACK - (OFF_B2 + 1),), jnp.float32)])
    out = _run(du, di, utf, itf, wpack)
    return out.reshape(BATCH, 1)
